# baseline (speedup 1.0000x reference)
import jax
import jax.numpy as jnp
from jax.experimental import pallas as pl
from jax.experimental.pallas import tpu as pltpu

N = 10000
TM = 200
NBUF = 4
FEA = 24
N_TILES = N // TM


def _xw_kernel(xall_ref, wcat_ref, xw_ref):
    xw_ref[...] = jnp.dot(xall_ref[...], wcat_ref[...],
                          preferred_element_type=jnp.float32)


def _pass1_kernel(adj_ref, xw_ref, bcat_ref,
                  wq_ref, wk_ref, wv_ref, bq_ref, bk_ref, bv_ref,
                  v_out_ref, s_out_ref, buf, sem):
    def make_copy(t):
        return pltpu.make_async_copy(
            adj_ref.at[pl.ds(t * TM, TM), :],
            buf.at[t % NBUF],
            sem.at[t % NBUF])

    for t in range(NBUF):
        make_copy(t).start()

    for t in range(N_TILES):
        make_copy(t).wait()
        acc = jnp.dot(buf[t % NBUF], xw_ref[...],
                      preferred_element_type=jnp.float32)
        if t + NBUF < N_TILES:
            make_copy(t + NBUF).start()
        x = jnp.maximum(acc + bcat_ref[...], 0.0)
        q = jnp.dot(x, wq_ref[...], preferred_element_type=jnp.float32) + bq_ref[...]
        k = jnp.dot(x, wk_ref[...], preferred_element_type=jnp.float32) + bk_ref[...]
        v = jnp.dot(x, wv_ref[...], preferred_element_type=jnp.float32) + bv_ref[...]
        v_out_ref[pl.ds(t * TM, TM), :] = v
        s = jax.lax.dot_general(q, k, (((0,), (0,)), ((), ())),
                                preferred_element_type=jnp.float32)
        s_out_ref[...] = s if t == 0 else s_out_ref[...] + s


def _pass2_kernel(v_ref, s_ref, x1_ref, x2_ref, x3_ref,
                  w1a_ref, w1b_ref, w1c_ref, w1d_ref, bl1_ref,
                  wl2_ref, bl2_ref, out_ref):
    s = s_ref[...] * (1.0 / (FEA ** 0.5))
    s = s - jnp.max(s, axis=-1, keepdims=True)
    e = jnp.exp(s)
    a = e / jnp.sum(e, axis=-1, keepdims=True)

    attn = jax.lax.dot_general(v_ref[...], a, (((1,), (1,)), ((), ())),
                               preferred_element_type=jnp.float32)

    h = (jnp.dot(x1_ref[...], w1a_ref[...], preferred_element_type=jnp.float32)
         + jnp.dot(x2_ref[...], w1b_ref[...], preferred_element_type=jnp.float32)
         + jnp.dot(x3_ref[...], w1c_ref[...], preferred_element_type=jnp.float32)
         + jnp.dot(attn, w1d_ref[...], preferred_element_type=jnp.float32)
         + bl1_ref[...])
    h = jnp.maximum(h, 0.0)
    out = jnp.dot(h, wl2_ref[...], preferred_element_type=jnp.float32) + bl2_ref[...]
    out_ref[...] = jnp.maximum(out, 0.0)


@jax.jit
def kernel(x1, x2, x3, adj, W1, b1, W2, b2, W3, b3, Wqkv, bqkv, Wl1, bl1, Wl2, bl2):
    f32 = jnp.float32
    xall = jnp.concatenate([x1, x2, x3], axis=1)
    wcat = jnp.zeros((60, 24), f32)
    wcat = wcat.at[0:20, 0:8].set(W1)
    wcat = wcat.at[20:40, 8:16].set(W2)
    wcat = wcat.at[40:60, 16:24].set(W3)
    bcat = jnp.concatenate([b1, b2, b3]).reshape(1, 24)

    wq = Wqkv[:, 0:24]
    wk = Wqkv[:, 24:48]
    wv = Wqkv[:, 48:72]
    bq = bqkv[0:24].reshape(1, 24)
    bk = bqkv[24:48].reshape(1, 24)
    bv = bqkv[48:72].reshape(1, 24)

    xw = pl.pallas_call(
        _xw_kernel,
        in_specs=[pl.BlockSpec((N, 60), lambda: (0, 0)),
                  pl.BlockSpec((60, 24), lambda: (0, 0))],
        out_specs=pl.BlockSpec((N, 24), lambda: (0, 0)),
        out_shape=jax.ShapeDtypeStruct((N, 24), f32),
    )(xall, wcat)

    vmem = lambda: pl.BlockSpec(memory_space=pltpu.MemorySpace.VMEM)
    v, s = pl.pallas_call(
        _pass1_kernel,
        in_specs=[
            pl.BlockSpec(memory_space=pltpu.MemorySpace.HBM),
            vmem(),
            vmem(),
            vmem(), vmem(), vmem(),
            vmem(), vmem(), vmem(),
        ],
        out_specs=[vmem(), vmem()],
        out_shape=[
            jax.ShapeDtypeStruct((N, 24), f32),
            jax.ShapeDtypeStruct((FEA, FEA), f32),
        ],
        scratch_shapes=[
            pltpu.VMEM((NBUF, TM, N), f32),
            pltpu.SemaphoreType.DMA((NBUF,)),
        ],
    )(adj, xw, bcat, wq, wk, wv, bq, bk, bv)

    w1a = Wl1[0:20]
    w1b = Wl1[20:40]
    w1c = Wl1[40:60]
    w1d = Wl1[60:84]

    num_tiles = N // TM
    const = lambda shape: pl.BlockSpec(shape, lambda i: (0, 0))
    row = lambda w: pl.BlockSpec((TM, w), lambda i: (i, 0))

    out = pl.pallas_call(
        _pass2_kernel,
        grid=(num_tiles,),
        in_specs=[
            row(24),
            const((FEA, FEA)),
            row(20), row(20), row(20),
            const((20, 16)), const((20, 16)), const((20, 16)), const((24, 16)),
            const((1, 16)),
            const((16, 7)),
            const((1, 7)),
        ],
        out_specs=row(7),
        out_shape=jax.ShapeDtypeStruct((N, 7), f32),
    )(v, s, x1, x2, x3, w1a, w1b, w1c, w1d,
      bl1.reshape(1, 16), Wl2, bl2.reshape(1, 7))
    return out


# speedup vs baseline: 1.2514x; 1.2514x over previous
import jax
import jax.numpy as jnp
from jax.experimental import pallas as pl
from jax.experimental.pallas import tpu as pltpu

N = 10000
TM = 400
TM2 = 2000
FEA = 24


def _pass1_kernel(adj_ref, xall_ref, wcat_ref, bcat_ref,
                  wq_ref, wk_ref, wv_ref, bq_ref, bk_ref, bv_ref,
                  v_out_ref, s_out_ref, xw_s):
    i = pl.program_id(0)

    @pl.when(i == 0)
    def _():
        xw_s[...] = jnp.dot(xall_ref[...], wcat_ref[...],
                            preferred_element_type=jnp.float32)

    acc = jnp.dot(adj_ref[...], xw_s[...], preferred_element_type=jnp.float32)
    x = jnp.maximum(acc + bcat_ref[...], 0.0)

    q = jnp.dot(x, wq_ref[...], preferred_element_type=jnp.float32) + bq_ref[...]
    k = jnp.dot(x, wk_ref[...], preferred_element_type=jnp.float32) + bk_ref[...]
    v = jnp.dot(x, wv_ref[...], preferred_element_type=jnp.float32) + bv_ref[...]
    v_out_ref[...] = v

    s = jax.lax.dot_general(q, k, (((0,), (0,)), ((), ())),
                            preferred_element_type=jnp.float32)

    @pl.when(i == 0)
    def _():
        s_out_ref[...] = s

    @pl.when(i != 0)
    def _():
        s_out_ref[...] = s_out_ref[...] + s


def _pass2_kernel(v_ref, s_ref, x1_ref, x2_ref, x3_ref,
                  w1a_ref, w1b_ref, w1c_ref, w1d_ref, bl1_ref,
                  wl2_ref, bl2_ref, out_ref):
    s = s_ref[...] * (1.0 / (FEA ** 0.5))
    s = s - jnp.max(s, axis=-1, keepdims=True)
    e = jnp.exp(s)
    a = e / jnp.sum(e, axis=-1, keepdims=True)

    attn = jax.lax.dot_general(v_ref[...], a, (((1,), (1,)), ((), ())),
                               preferred_element_type=jnp.float32)

    h = (jnp.dot(x1_ref[...], w1a_ref[...], preferred_element_type=jnp.float32)
         + jnp.dot(x2_ref[...], w1b_ref[...], preferred_element_type=jnp.float32)
         + jnp.dot(x3_ref[...], w1c_ref[...], preferred_element_type=jnp.float32)
         + jnp.dot(attn, w1d_ref[...], preferred_element_type=jnp.float32)
         + bl1_ref[...])
    h = jnp.maximum(h, 0.0)
    out = jnp.dot(h, wl2_ref[...], preferred_element_type=jnp.float32) + bl2_ref[...]
    out_ref[...] = jnp.maximum(out, 0.0)


@jax.jit
def kernel(x1, x2, x3, adj, W1, b1, W2, b2, W3, b3, Wqkv, bqkv, Wl1, bl1, Wl2, bl2):
    f32 = jnp.float32
    xall = jnp.concatenate([x1, x2, x3], axis=1)
    wcat = jnp.zeros((60, 24), f32)
    wcat = wcat.at[0:20, 0:8].set(W1)
    wcat = wcat.at[20:40, 8:16].set(W2)
    wcat = wcat.at[40:60, 16:24].set(W3)
    bcat = jnp.concatenate([b1, b2, b3]).reshape(1, 24)

    wq = Wqkv[:, 0:24]
    wk = Wqkv[:, 24:48]
    wv = Wqkv[:, 48:72]
    bq = bqkv[0:24].reshape(1, 24)
    bk = bqkv[24:48].reshape(1, 24)
    bv = bqkv[48:72].reshape(1, 24)

    const = lambda shape: pl.BlockSpec(shape, lambda i: (0, 0))
    row1 = lambda w: pl.BlockSpec((TM, w), lambda i: (i, 0))
    row2 = lambda w: pl.BlockSpec((TM2, w), lambda i: (i, 0))

    v, s = pl.pallas_call(
        _pass1_kernel,
        grid=(N // TM,),
        in_specs=[
            row1(N),
            const((N, 60)),
            const((60, 24)),
            const((1, 24)),
            const((24, 24)), const((24, 24)), const((24, 24)),
            const((1, 24)), const((1, 24)), const((1, 24)),
        ],
        out_specs=[
            row1(24),
            pl.BlockSpec((FEA, FEA), lambda i: (0, 0)),
        ],
        out_shape=[
            jax.ShapeDtypeStruct((N, 24), f32),
            jax.ShapeDtypeStruct((FEA, FEA), f32),
        ],
        scratch_shapes=[pltpu.VMEM((N, 24), f32)],
    )(adj, xall, wcat, bcat, wq, wk, wv, bq, bk, bv)

    w1a = Wl1[0:20]
    w1b = Wl1[20:40]
    w1c = Wl1[40:60]
    w1d = Wl1[60:84]

    out = pl.pallas_call(
        _pass2_kernel,
        grid=(N // TM2,),
        in_specs=[
            row2(24),
            const((FEA, FEA)),
            row2(20), row2(20), row2(20),
            const((20, 16)), const((20, 16)), const((20, 16)), const((24, 16)),
            const((1, 16)),
            const((16, 7)),
            const((1, 7)),
        ],
        out_specs=row2(7),
        out_shape=jax.ShapeDtypeStruct((N, 7), f32),
    )(v, s, x1, x2, x3, w1a, w1b, w1c, w1d,
      bl1.reshape(1, 16), Wl2, bl2.reshape(1, 7))
    return out
